# baseline (speedup 1.0000x reference)
import functools

import jax
import jax.numpy as jnp
from jax import lax
from jax.experimental import pallas as pl
from jax.experimental.pallas import tpu as pltpu
from jax.experimental.pallas import tpu_sc as plsc

B = 16384
DIM = 32
NC, NS = 2, 16
NW = NC * NS
B_PER_W = B // NW
E_PER_W = B_PER_W * DIM
MLP_BLOCK = 2048


def _sc_gather2(uflat, rflat, uidx, ridx):
    @functools.partial(
        pl.kernel,
        mesh=plsc.VectorSubcoreMesh(core_axis_name="c", subcore_axis_name="s"),
        out_type=(
            jax.ShapeDtypeStruct((B * DIM,), jnp.float32),
            jax.ShapeDtypeStruct((B * DIM,), jnp.float32),
        ),
        scratch_types=[
            pltpu.VMEM((E_PER_W,), jnp.int32),
            pltpu.VMEM((E_PER_W,), jnp.int32),
            pltpu.VMEM((E_PER_W,), jnp.float32),
            pltpu.VMEM((E_PER_W,), jnp.float32),
            pltpu.SemaphoreType.DMA,
            pltpu.SemaphoreType.DMA,
        ],
    )
    def k(uf_hbm, rf_hbm, ui_hbm, ri_hbm, uo_hbm, ro_hbm,
          uidx_v, ridx_v, uval_v, rval_v, usem, rsem):
        wid = lax.axis_index("s") * NC + lax.axis_index("c")
        base = wid * E_PER_W
        pltpu.sync_copy(ui_hbm.at[pl.ds(base, E_PER_W)], uidx_v)
        pltpu.sync_copy(ri_hbm.at[pl.ds(base, E_PER_W)], ridx_v)
        ucp = pltpu.async_copy(uf_hbm.at[uidx_v], uval_v, usem)
        rcp = pltpu.async_copy(rf_hbm.at[ridx_v], rval_v, rsem)
        ucp.wait()
        rcp.wait()
        pltpu.sync_copy(uval_v, uo_hbm.at[pl.ds(base, E_PER_W)])
        pltpu.sync_copy(rval_v, ro_hbm.at[pl.ds(base, E_PER_W)])

    return k(uflat, rflat, uidx, ridx)


def _mlp_body(u_ref, r_ref, w1u_ref, w1r_ref, b1_ref, w2_ref, b2_ref,
              w3_ref, b3_ref, o_ref):
    h = jnp.dot(u_ref[...], w1u_ref[...], preferred_element_type=jnp.float32)
    h += jnp.dot(r_ref[...], w1r_ref[...], preferred_element_type=jnp.float32)
    h = jnp.maximum(h + b1_ref[...], 0.0)
    h = jnp.dot(h, w2_ref[...], preferred_element_type=jnp.float32)
    h = jnp.maximum(h + b2_ref[...], 0.0)
    o_ref[...] = (
        jnp.dot(h, w3_ref[...], preferred_element_type=jnp.float32)
        + b3_ref[...]
    )


def _tc_mlp(u, r, W1, b1, W2, b2, W3, b3):
    w1u = W1[:DIM]
    w1r = W1[DIM:]
    b1r = b1.reshape(1, -1)
    b2r = b2.reshape(1, -1)
    b3r = b3.reshape(1, -1)
    const = lambda shape: pl.BlockSpec(shape, lambda i: (0, 0))
    return pl.pallas_call(
        _mlp_body,
        grid=(B // MLP_BLOCK,),
        in_specs=[
            pl.BlockSpec((MLP_BLOCK, DIM), lambda i: (i, 0)),
            pl.BlockSpec((MLP_BLOCK, DIM), lambda i: (i, 0)),
            const(w1u.shape),
            const(w1r.shape),
            const(b1r.shape),
            const(W2.shape),
            const(b2r.shape),
            const(W3.shape),
            const(b3r.shape),
        ],
        out_specs=pl.BlockSpec((MLP_BLOCK, 1), lambda i: (i, 0)),
        out_shape=jax.ShapeDtypeStruct((B, 1), jnp.float32),
    )(u, r, w1u, w1r, b1r, W2, b2r, W3, b3r)


def kernel(user_id, recipe_id, user_table, recipe_table, W1, b1, W2, b2, W3, b3):
    uv = user_table.shape[0]
    rv = recipe_table.shape[0]
    uflat = user_table.T.reshape(-1)
    rflat = recipe_table.T.reshape(-1)
    feat = jnp.arange(DIM, dtype=jnp.int32)[None, :]
    uidx = (feat * uv + user_id.astype(jnp.int32)[:, None]).reshape(-1)
    ridx = (feat * rv + recipe_id.astype(jnp.int32)[:, None]).reshape(-1)
    uo, ro = _sc_gather2(uflat, rflat, uidx, ridx)
    u = uo.reshape(B, DIM)
    r = ro.reshape(B, DIM)
    return _tc_mlp(u, r, W1, b1, W2, b2, W3, b3)


# speedup vs baseline: 11.9907x; 11.9907x over previous
import functools

import jax
import jax.numpy as jnp
from jax import lax
from jax.experimental import pallas as pl
from jax.experimental.pallas import tpu as pltpu
from jax.experimental.pallas import tpu_sc as plsc

B = 16384
DIM = 32
NC, NS = 2, 16
NW = NC * NS
B_PER_W = B // NW
E_PER_W = B_PER_W * DIM
MLP_BLOCK = 2048

UV = 1000001
RV = 100001
UPT = 7813
RPT = 782
UK = 601
RK = 391


def _flatten_body(in_ref, out_ref):
    out_ref[...] = pltpu.einshape(
        "s(tc)->(ts)c", in_ref[...], c=128, assert_is_tile_preserving=True
    )


def _flatten(tableT, pt, k):
    nfeat_blocks = DIM // 8
    out_rows = nfeat_blocks * 8 * pt
    return pl.pallas_call(
        _flatten_body,
        grid=(nfeat_blocks, pt // k),
        in_specs=[pl.BlockSpec((8, k * 128), lambda c8, j: (c8, j))],
        out_specs=pl.BlockSpec((k * 8, 128), lambda c8, j, _pt=pt, _k=k:
                               (c8 * (_pt // _k) + j, 0)),
        out_shape=jax.ShapeDtypeStruct((out_rows, 128), jnp.float32),
    )(tableT)


def _sc_gather2(uflat, rflat, uidx, ridx):
    @functools.partial(
        pl.kernel,
        mesh=plsc.VectorSubcoreMesh(core_axis_name="c", subcore_axis_name="s"),
        out_type=(
            jax.ShapeDtypeStruct((B * DIM,), jnp.float32),
            jax.ShapeDtypeStruct((B * DIM,), jnp.float32),
        ),
        scratch_types=[
            pltpu.VMEM((E_PER_W,), jnp.int32),
            pltpu.VMEM((E_PER_W,), jnp.int32),
            pltpu.VMEM((E_PER_W,), jnp.float32),
            pltpu.VMEM((E_PER_W,), jnp.float32),
            pltpu.SemaphoreType.DMA,
            pltpu.SemaphoreType.DMA,
        ],
    )
    def k(uf_hbm, rf_hbm, ui_hbm, ri_hbm, uo_hbm, ro_hbm,
          uidx_v, ridx_v, uval_v, rval_v, usem, rsem):
        wid = lax.axis_index("s") * NC + lax.axis_index("c")
        base = wid * E_PER_W
        pltpu.sync_copy(ui_hbm.at[pl.ds(base, E_PER_W)], uidx_v)
        pltpu.sync_copy(ri_hbm.at[pl.ds(base, E_PER_W)], ridx_v)
        ucp = pltpu.async_copy(uf_hbm.at[uidx_v], uval_v, usem)
        rcp = pltpu.async_copy(rf_hbm.at[ridx_v], rval_v, rsem)
        ucp.wait()
        rcp.wait()
        pltpu.sync_copy(uval_v, uo_hbm.at[pl.ds(base, E_PER_W)])
        pltpu.sync_copy(rval_v, ro_hbm.at[pl.ds(base, E_PER_W)])

    return k(uflat, rflat, uidx, ridx)


def _mlp_body(u_ref, r_ref, w1u_ref, w1r_ref, b1_ref, w2_ref, b2_ref,
              w3_ref, b3_ref, o_ref):
    h = jnp.dot(u_ref[...], w1u_ref[...], preferred_element_type=jnp.float32)
    h += jnp.dot(r_ref[...], w1r_ref[...], preferred_element_type=jnp.float32)
    h = jnp.maximum(h + b1_ref[...], 0.0)
    h = jnp.dot(h, w2_ref[...], preferred_element_type=jnp.float32)
    h = jnp.maximum(h + b2_ref[...], 0.0)
    o_ref[...] = (
        jnp.dot(h, w3_ref[...], preferred_element_type=jnp.float32)
        + b3_ref[...]
    )


def _tc_mlp(u, r, W1, b1, W2, b2, W3, b3):
    w1u = W1[:DIM]
    w1r = W1[DIM:]
    b1r = b1.reshape(1, -1)
    b2r = b2.reshape(1, -1)
    b3r = b3.reshape(1, -1)
    const = lambda shape: pl.BlockSpec(shape, lambda i: (0, 0))
    return pl.pallas_call(
        _mlp_body,
        grid=(B // MLP_BLOCK,),
        in_specs=[
            pl.BlockSpec((MLP_BLOCK, DIM), lambda i: (i, 0)),
            pl.BlockSpec((MLP_BLOCK, DIM), lambda i: (i, 0)),
            const(w1u.shape),
            const(w1r.shape),
            const(b1r.shape),
            const(W2.shape),
            const(b2r.shape),
            const(W3.shape),
            const(b3r.shape),
        ],
        out_specs=pl.BlockSpec((MLP_BLOCK, 1), lambda i: (i, 0)),
        out_shape=jax.ShapeDtypeStruct((B, 1), jnp.float32),
    )(u, r, w1u, w1r, b1r, W2, b2r, W3, b3r)


def _tile_order_idx(ids, pt):
    c = jnp.arange(DIM, dtype=jnp.int32)[None, :]
    cterm = (c // 8) * (pt * 1024) + (c % 8) * 128
    v = ids.astype(jnp.int32)[:, None]
    vterm = (v // 128) * 1024 + v % 128
    return (cterm + vterm).reshape(-1)


def kernel(user_id, recipe_id, user_table, recipe_table, W1, b1, W2, b2, W3, b3):
    uflat = _flatten(user_table.T, UPT, UK).reshape(-1)
    rflat = _flatten(recipe_table.T, RPT, RK).reshape(-1)
    uidx = _tile_order_idx(user_id, UPT)
    ridx = _tile_order_idx(recipe_id, RPT)
    uo, ro = _sc_gather2(uflat, rflat, uidx, ridx)
    u = uo.reshape(B, DIM)
    r = ro.reshape(B, DIM)
    return _tc_mlp(u, r, W1, b1, W2, b2, W3, b3)


# speedup vs baseline: 12.9322x; 1.0785x over previous
import functools

import jax
import jax.numpy as jnp
from jax import lax
from jax.experimental import pallas as pl
from jax.experimental.pallas import tpu as pltpu
from jax.experimental.pallas import tpu_sc as plsc

B = 16384
DIM = 32
NC, NS = 2, 16
NW = NC * NS
B_PER_W = B // NW
E_PER_W = B_PER_W * DIM
MLP_BLOCK = 2048

UV = 1000001
RV = 100001
UPT = 7813
RPT = 782
UK = 601
RK = 391


def _flatten_body(in_ref, out_ref):
    out_ref[...] = pltpu.einshape(
        "s(tc)->(ts)c", in_ref[...], c=128, assert_is_tile_preserving=True
    )


def _flatten(tableT, pt, k):
    nfeat_blocks = DIM // 8
    out_rows = nfeat_blocks * 8 * pt
    return pl.pallas_call(
        _flatten_body,
        grid=(nfeat_blocks, pt // k),
        in_specs=[pl.BlockSpec((8, k * 128), lambda c8, j: (c8, j))],
        out_specs=pl.BlockSpec((k * 8, 128), lambda c8, j, _pt=pt, _k=k:
                               (c8 * (_pt // _k) + j, 0)),
        out_shape=jax.ShapeDtypeStruct((out_rows, 128), jnp.float32),
        compiler_params=pltpu.CompilerParams(
            dimension_semantics=("parallel", "arbitrary")),
    )(tableT)


def _sc_gather(flat, idx):
    @functools.partial(
        pl.kernel,
        mesh=plsc.VectorSubcoreMesh(core_axis_name="c", subcore_axis_name="s"),
        out_type=jax.ShapeDtypeStruct((B * DIM,), jnp.float32),
        scratch_types=[
            pltpu.VMEM((E_PER_W,), jnp.int32),
            pltpu.VMEM((E_PER_W,), jnp.float32),
            pltpu.SemaphoreType.DMA,
        ],
    )
    def k(f_hbm, i_hbm, o_hbm, idx_v, val_v, sem):
        wid = lax.axis_index("s") * NC + lax.axis_index("c")
        base = wid * E_PER_W
        pltpu.sync_copy(i_hbm.at[pl.ds(base, E_PER_W)], idx_v)
        pltpu.async_copy(f_hbm.at[idx_v], val_v, sem).wait()
        pltpu.sync_copy(val_v, o_hbm.at[pl.ds(base, E_PER_W)])

    return k(flat, idx)


def _mlp_body(u_ref, r_ref, w1u_ref, w1r_ref, b1_ref, w2_ref, b2_ref,
              w3_ref, b3_ref, o_ref):
    h = jnp.dot(u_ref[...], w1u_ref[...], preferred_element_type=jnp.float32)
    h += jnp.dot(r_ref[...], w1r_ref[...], preferred_element_type=jnp.float32)
    h = jnp.maximum(h + b1_ref[...], 0.0)
    h = jnp.dot(h, w2_ref[...], preferred_element_type=jnp.float32)
    h = jnp.maximum(h + b2_ref[...], 0.0)
    o_ref[...] = (
        jnp.dot(h, w3_ref[...], preferred_element_type=jnp.float32)
        + b3_ref[...]
    )


def _tc_mlp(u, r, W1, b1, W2, b2, W3, b3):
    w1u = W1[:DIM]
    w1r = W1[DIM:]
    b1r = b1.reshape(1, -1)
    b2r = b2.reshape(1, -1)
    b3r = b3.reshape(1, -1)
    const = lambda shape: pl.BlockSpec(shape, lambda i: (0, 0))
    return pl.pallas_call(
        _mlp_body,
        grid=(B // MLP_BLOCK,),
        in_specs=[
            pl.BlockSpec((MLP_BLOCK, DIM), lambda i: (i, 0)),
            pl.BlockSpec((MLP_BLOCK, DIM), lambda i: (i, 0)),
            const(w1u.shape),
            const(w1r.shape),
            const(b1r.shape),
            const(W2.shape),
            const(b2r.shape),
            const(W3.shape),
            const(b3r.shape),
        ],
        out_specs=pl.BlockSpec((MLP_BLOCK, 1), lambda i: (i, 0)),
        out_shape=jax.ShapeDtypeStruct((B, 1), jnp.float32),
        compiler_params=pltpu.CompilerParams(
            dimension_semantics=("parallel",)),
    )(u, r, w1u, w1r, b1r, W2, b2r, W3, b3r)


def _tile_order_idx(ids, pt):
    c = jnp.arange(DIM, dtype=jnp.int32)[None, :]
    cterm = (c // 8) * (pt * 1024) + (c % 8) * 128
    v = ids.astype(jnp.int32)[:, None]
    vterm = (v // 128) * 1024 + v % 128
    return (cterm + vterm).reshape(-1)


def kernel(user_id, recipe_id, user_table, recipe_table, W1, b1, W2, b2, W3, b3):
    rflat = _flatten(recipe_table.T, RPT, RK).reshape(-1)
    uflat = _flatten(user_table.T, UPT, UK).reshape(-1)
    uidx = _tile_order_idx(user_id, UPT)
    ridx = _tile_order_idx(recipe_id, RPT)
    ro = _sc_gather(rflat, ridx)
    uo = _sc_gather(uflat, uidx)
    u = uo.reshape(B, DIM)
    r = ro.reshape(B, DIM)
    return _tc_mlp(u, r, W1, b1, W2, b2, W3, b3)


# speedup vs baseline: 15.8937x; 1.2290x over previous
import functools

import jax
import jax.numpy as jnp
from jax import lax
from jax.experimental import pallas as pl
from jax.experimental.pallas import tpu as pltpu
from jax.experimental.pallas import tpu_sc as plsc

B = 16384
DIM = 32
PAIRS = DIM // 2
NC, NS = 2, 16
NW = NC * NS
B_PER_W = B // NW
E_PER_W = B_PER_W * PAIRS
MLP_BLOCK = 2048

UV = 1000001
RV = 100001
UPT = 7813
RPT = 782
UK = 601
RK = 391

PERM_LO = [16 * (i // 8) + i % 8 for i in range(PAIRS)]
PERM_HI = [16 * (i // 8) + i % 8 + 8 for i in range(PAIRS)]


def _flatten_pack_body(in_ref, out_ref):
    lo = in_ref[:8, :].astype(jnp.bfloat16)
    hi = in_ref[8:, :].astype(jnp.bfloat16)
    lo_u = lax.bitcast_convert_type(lo, jnp.uint16).astype(jnp.uint32)
    hi_u = lax.bitcast_convert_type(hi, jnp.uint16).astype(jnp.uint32)
    packed = lax.bitcast_convert_type(lo_u | (hi_u << 16), jnp.float32)
    out_ref[...] = pltpu.einshape(
        "s(tc)->(ts)c", packed, c=128, assert_is_tile_preserving=True
    )


def _flatten_pack(tableT, pt, k):
    ngroups = DIM // 16
    out_rows = ngroups * 8 * pt
    return pl.pallas_call(
        _flatten_pack_body,
        grid=(ngroups, pt // k),
        in_specs=[pl.BlockSpec((16, k * 128), lambda g, j: (g, j))],
        out_specs=pl.BlockSpec((k * 8, 128), lambda g, j, _pt=pt, _k=k:
                               (g * (_pt // _k) + j, 0)),
        out_shape=jax.ShapeDtypeStruct((out_rows, 128), jnp.float32),
        compiler_params=pltpu.CompilerParams(
            dimension_semantics=("parallel", "arbitrary")),
    )(tableT)


def _sc_gather(flat, idx):
    @functools.partial(
        pl.kernel,
        mesh=plsc.VectorSubcoreMesh(core_axis_name="c", subcore_axis_name="s"),
        out_type=jax.ShapeDtypeStruct((B * PAIRS,), jnp.float32),
        scratch_types=[
            pltpu.VMEM((E_PER_W,), jnp.int32),
            pltpu.VMEM((E_PER_W,), jnp.float32),
            pltpu.SemaphoreType.DMA,
        ],
    )
    def k(f_hbm, i_hbm, o_hbm, idx_v, val_v, sem):
        wid = lax.axis_index("s") * NC + lax.axis_index("c")
        base = wid * E_PER_W
        pltpu.sync_copy(i_hbm.at[pl.ds(base, E_PER_W)], idx_v)
        pltpu.async_copy(f_hbm.at[idx_v], val_v, sem).wait()
        pltpu.sync_copy(val_v, o_hbm.at[pl.ds(base, E_PER_W)])

    return k(flat, idx)


def _unpack(p_u32):
    lo = lax.bitcast_convert_type(p_u32 << 16, jnp.float32)
    hi = lax.bitcast_convert_type(p_u32 & jnp.uint32(0xFFFF0000), jnp.float32)
    return lo.astype(jnp.bfloat16), hi.astype(jnp.bfloat16)


def _mlp_body(u_ref, r_ref, w1ul_ref, w1uh_ref, w1rl_ref, w1rh_ref, b1_ref,
              w2_ref, b2_ref, w3_ref, b3_ref, o_ref):
    u_lo, u_hi = _unpack(lax.bitcast_convert_type(u_ref[...], jnp.uint32))
    r_lo, r_hi = _unpack(lax.bitcast_convert_type(r_ref[...], jnp.uint32))
    f32 = jnp.float32
    h = jnp.dot(u_lo, w1ul_ref[...], preferred_element_type=f32)
    h += jnp.dot(u_hi, w1uh_ref[...], preferred_element_type=f32)
    h += jnp.dot(r_lo, w1rl_ref[...], preferred_element_type=f32)
    h += jnp.dot(r_hi, w1rh_ref[...], preferred_element_type=f32)
    h = jnp.maximum(h + b1_ref[...], 0.0)
    h = jnp.dot(h, w2_ref[...], preferred_element_type=f32)
    h = jnp.maximum(h + b2_ref[...], 0.0)
    o_ref[...] = (
        jnp.dot(h, w3_ref[...], preferred_element_type=f32) + b3_ref[...]
    )


def _tc_mlp(u, r, W1, b1, W2, b2, W3, b3):
    bf16 = jnp.bfloat16
    w1u = W1[:DIM]
    w1r = W1[DIM:]
    w1ul = w1u[jnp.array(PERM_LO)].astype(bf16)
    w1uh = w1u[jnp.array(PERM_HI)].astype(bf16)
    w1rl = w1r[jnp.array(PERM_LO)].astype(bf16)
    w1rh = w1r[jnp.array(PERM_HI)].astype(bf16)
    b1r = b1.reshape(1, -1)
    b2r = b2.reshape(1, -1)
    b3r = b3.reshape(1, -1)
    const = lambda shape: pl.BlockSpec(shape, lambda i: (0, 0))
    return pl.pallas_call(
        _mlp_body,
        grid=(B // MLP_BLOCK,),
        in_specs=[
            pl.BlockSpec((MLP_BLOCK, PAIRS), lambda i: (i, 0)),
            pl.BlockSpec((MLP_BLOCK, PAIRS), lambda i: (i, 0)),
            const(w1ul.shape),
            const(w1uh.shape),
            const(w1rl.shape),
            const(w1rh.shape),
            const(b1r.shape),
            const(W2.shape),
            const(b2r.shape),
            const(W3.shape),
            const(b3r.shape),
        ],
        out_specs=pl.BlockSpec((MLP_BLOCK, 1), lambda i: (i, 0)),
        out_shape=jax.ShapeDtypeStruct((B, 1), jnp.float32),
        compiler_params=pltpu.CompilerParams(
            dimension_semantics=("parallel",)),
    )(u, r, w1ul, w1uh, w1rl, w1rh, b1r, W2, b2r, W3, b3r)


def _pair_idx(ids, pt):
    i = jnp.arange(PAIRS, dtype=jnp.int32)[None, :]
    gk = (i // 8) * (pt * 1024) + (i % 8) * 128
    v = ids.astype(jnp.int32)[:, None]
    vterm = (v // 128) * 1024 + v % 128
    return (gk + vterm).reshape(-1)


def kernel(user_id, recipe_id, user_table, recipe_table, W1, b1, W2, b2, W3, b3):
    ridx = _pair_idx(recipe_id, RPT)
    uidx = _pair_idx(user_id, UPT)
    rflat = _flatten_pack(recipe_table.T, RPT, RK).reshape(-1)
    user_tableT, _ = lax.optimization_barrier((user_table.T, (rflat, ridx)))
    uflat = _flatten_pack(user_tableT, UPT, UK).reshape(-1)
    ro = _sc_gather(rflat, ridx)
    uo = _sc_gather(uflat, uidx)
    u = uo.reshape(B, PAIRS)
    r = ro.reshape(B, PAIRS)
    return _tc_mlp(u, r, W1, b1, W2, b2, W3, b3)


# speedup vs baseline: 17.1128x; 1.0767x over previous
import functools

import jax
import jax.numpy as jnp
from jax import lax
from jax.experimental import pallas as pl
from jax.experimental.pallas import tpu as pltpu
from jax.experimental.pallas import tpu_sc as plsc

B = 16384
DIM = 32
PAIRS = DIM // 2
NC, NS = 2, 16
NW = NC * NS
B_PER_W = B // NW
E_PER_W = B_PER_W * PAIRS
MLP_BLOCK = 2048

UV = 1000001
RV = 100001
UPT = 7813
RPT = 782
UK = 601
RK = 391

PERM_LO = [16 * (i // 8) + i % 8 for i in range(PAIRS)]
PERM_HI = [16 * (i // 8) + i % 8 + 8 for i in range(PAIRS)]


def _flatten_pack_body(in_ref, out_ref):
    lo = in_ref[:8, :].astype(jnp.bfloat16)
    hi = in_ref[8:, :].astype(jnp.bfloat16)
    lo_u = lax.bitcast_convert_type(lo, jnp.uint16).astype(jnp.uint32)
    hi_u = lax.bitcast_convert_type(hi, jnp.uint16).astype(jnp.uint32)
    packed = lax.bitcast_convert_type(lo_u | (hi_u << 16), jnp.float32)
    out_ref[...] = pltpu.einshape(
        "s(tc)->(ts)c", packed, c=128, assert_is_tile_preserving=True
    )


def _flatten_pack(tableT, pt, k):
    ngroups = DIM // 16
    out_rows = ngroups * 8 * pt
    return pl.pallas_call(
        _flatten_pack_body,
        grid=(ngroups, pt // k),
        in_specs=[pl.BlockSpec((16, k * 128), lambda g, j: (g, j))],
        out_specs=pl.BlockSpec((k * 8, 128), lambda g, j, _pt=pt, _k=k:
                               (g * (_pt // _k) + j, 0)),
        out_shape=jax.ShapeDtypeStruct((out_rows, 128), jnp.float32),
        compiler_params=pltpu.CompilerParams(
            dimension_semantics=("parallel", "arbitrary")),
    )(tableT)


def _sc_gather(flat, ids, pt):
    gk = jnp.array(
        [(i // 8) * (pt * 1024) + (i % 8) * 128 for i in range(PAIRS)],
        dtype=jnp.int32,
    )

    @functools.partial(
        pl.kernel,
        mesh=plsc.VectorSubcoreMesh(core_axis_name="c", subcore_axis_name="s"),
        out_type=jax.ShapeDtypeStruct((B * PAIRS,), jnp.float32),
        compiler_params=pltpu.CompilerParams(needs_layout_passes=False),
        scratch_types=[
            pltpu.VMEM((B_PER_W,), jnp.int32),
            pltpu.VMEM((PAIRS,), jnp.int32),
            pltpu.VMEM((E_PER_W,), jnp.int32),
            pltpu.VMEM((E_PER_W,), jnp.float32),
            pltpu.SemaphoreType.DMA,
        ],
    )
    def k(f_hbm, ids_hbm, gk_hbm, o_hbm, ids_v, gk_v, idx_v, val_v, sem):
        wid = lax.axis_index("s") * NC + lax.axis_index("c")
        base = wid * B_PER_W
        pltpu.sync_copy(ids_hbm.at[pl.ds(base, B_PER_W)], ids_v)
        pltpu.sync_copy(gk_hbm, gk_v)
        gk_row = gk_v[...]

        @pl.loop(0, B_PER_W)
        def _(j):
            vj = plsc.load_gather(
                ids_v, [jnp.full((PAIRS,), j, jnp.int32)])
            vterm = ((vj >> 7) << 10) + (vj & 127)
            idx_v[pl.ds(j * PAIRS, PAIRS)] = gk_row + vterm

        pltpu.async_copy(f_hbm.at[idx_v], val_v, sem).wait()
        pltpu.sync_copy(val_v, o_hbm.at[pl.ds(wid * E_PER_W, E_PER_W)])

    return k(flat, ids, gk)


def _unpack(p_u32):
    lo = lax.bitcast_convert_type(p_u32 << 16, jnp.float32)
    hi = lax.bitcast_convert_type(p_u32 & jnp.uint32(0xFFFF0000), jnp.float32)
    return lo.astype(jnp.bfloat16), hi.astype(jnp.bfloat16)


def _mlp_body(u_ref, r_ref, w1ul_ref, w1uh_ref, w1rl_ref, w1rh_ref, b1_ref,
              w2_ref, b2_ref, w3_ref, b3_ref, o_ref):
    u_lo, u_hi = _unpack(lax.bitcast_convert_type(u_ref[...], jnp.uint32))
    r_lo, r_hi = _unpack(lax.bitcast_convert_type(r_ref[...], jnp.uint32))
    f32 = jnp.float32
    h = jnp.dot(u_lo, w1ul_ref[...], preferred_element_type=f32)
    h += jnp.dot(u_hi, w1uh_ref[...], preferred_element_type=f32)
    h += jnp.dot(r_lo, w1rl_ref[...], preferred_element_type=f32)
    h += jnp.dot(r_hi, w1rh_ref[...], preferred_element_type=f32)
    h = jnp.maximum(h + b1_ref[...], 0.0)
    h = jnp.dot(h, w2_ref[...], preferred_element_type=f32)
    h = jnp.maximum(h + b2_ref[...], 0.0)
    o_ref[...] = (
        jnp.dot(h, w3_ref[...], preferred_element_type=f32) + b3_ref[...]
    )


def _tc_mlp(u, r, W1, b1, W2, b2, W3, b3):
    bf16 = jnp.bfloat16
    w1u = W1[:DIM]
    w1r = W1[DIM:]
    w1ul = w1u[jnp.array(PERM_LO)].astype(bf16)
    w1uh = w1u[jnp.array(PERM_HI)].astype(bf16)
    w1rl = w1r[jnp.array(PERM_LO)].astype(bf16)
    w1rh = w1r[jnp.array(PERM_HI)].astype(bf16)
    b1r = b1.reshape(1, -1)
    b2r = b2.reshape(1, -1)
    b3r = b3.reshape(1, -1)
    const = lambda shape: pl.BlockSpec(shape, lambda i: (0, 0))
    return pl.pallas_call(
        _mlp_body,
        grid=(B // MLP_BLOCK,),
        in_specs=[
            pl.BlockSpec((MLP_BLOCK, PAIRS), lambda i: (i, 0)),
            pl.BlockSpec((MLP_BLOCK, PAIRS), lambda i: (i, 0)),
            const(w1ul.shape),
            const(w1uh.shape),
            const(w1rl.shape),
            const(w1rh.shape),
            const(b1r.shape),
            const(W2.shape),
            const(b2r.shape),
            const(W3.shape),
            const(b3r.shape),
        ],
        out_specs=pl.BlockSpec((MLP_BLOCK, 1), lambda i: (i, 0)),
        out_shape=jax.ShapeDtypeStruct((B, 1), jnp.float32),
        compiler_params=pltpu.CompilerParams(
            dimension_semantics=("parallel",)),
    )(u, r, w1ul, w1uh, w1rl, w1rh, b1r, W2, b2r, W3, b3r)


def _pair_idx(ids, pt):
    i = jnp.arange(PAIRS, dtype=jnp.int32)[None, :]
    gk = (i // 8) * (pt * 1024) + (i % 8) * 128
    v = ids.astype(jnp.int32)[:, None]
    vterm = (v // 128) * 1024 + v % 128
    return (gk + vterm).reshape(-1)


def kernel(user_id, recipe_id, user_table, recipe_table, W1, b1, W2, b2, W3, b3):
    rflat = _flatten_pack(recipe_table.T, RPT, RK).reshape(-1)
    user_tableT, _ = lax.optimization_barrier((user_table.T, rflat))
    uflat = _flatten_pack(user_tableT, UPT, UK).reshape(-1)
    ro = _sc_gather(rflat, recipe_id.astype(jnp.int32), RPT)
    uo = _sc_gather(uflat, user_id.astype(jnp.int32), UPT)
    u = uo.reshape(B, PAIRS)
    r = ro.reshape(B, PAIRS)
    return _tc_mlp(u, r, W1, b1, W2, b2, W3, b3)


# speedup vs baseline: 17.1236x; 1.0006x over previous
import functools

import jax
import jax.numpy as jnp
from jax import lax
from jax.experimental import pallas as pl
from jax.experimental.pallas import tpu as pltpu
from jax.experimental.pallas import tpu_sc as plsc

B = 16384
DIM = 32
PAIRS = DIM // 2
NC, NS = 2, 16
NW = NC * NS
B_PER_W = B // NW
E_PER_W = B_PER_W * PAIRS
MLP_BLOCK = 4096

UV = 1000001
RV = 100001
UPT = 7813
RPT = 782
UK = 601
RK = 391

PERM_LO = [16 * (i // 8) + i % 8 for i in range(PAIRS)]
PERM_HI = [16 * (i // 8) + i % 8 + 8 for i in range(PAIRS)]


def _flatten_pack_body(in_ref, out_ref):
    lo = in_ref[:8, :].astype(jnp.bfloat16)
    hi = in_ref[8:, :].astype(jnp.bfloat16)
    lo_u = lax.bitcast_convert_type(lo, jnp.uint16).astype(jnp.uint32)
    hi_u = lax.bitcast_convert_type(hi, jnp.uint16).astype(jnp.uint32)
    packed = lax.bitcast_convert_type(lo_u | (hi_u << 16), jnp.float32)
    out_ref[...] = pltpu.einshape(
        "s(tc)->(ts)c", packed, c=128, assert_is_tile_preserving=True
    )


def _flatten_pack(tableT, pt, k):
    ngroups = DIM // 16
    out_rows = ngroups * 8 * pt
    return pl.pallas_call(
        _flatten_pack_body,
        grid=(ngroups, pt // k),
        in_specs=[pl.BlockSpec((16, k * 128), lambda g, j: (g, j))],
        out_specs=pl.BlockSpec((k * 8, 128), lambda g, j, _pt=pt, _k=k:
                               (g * (_pt // _k) + j, 0)),
        out_shape=jax.ShapeDtypeStruct((out_rows, 128), jnp.float32),
        compiler_params=pltpu.CompilerParams(
            dimension_semantics=("parallel", "arbitrary")),
    )(tableT)


def _sc_gather(flat, ids, pt):
    gk = jnp.array(
        [(i // 8) * (pt * 1024) + (i % 8) * 128 for i in range(PAIRS)],
        dtype=jnp.int32,
    )

    @functools.partial(
        pl.kernel,
        mesh=plsc.VectorSubcoreMesh(core_axis_name="c", subcore_axis_name="s"),
        out_type=jax.ShapeDtypeStruct((B * PAIRS,), jnp.float32),
        compiler_params=pltpu.CompilerParams(needs_layout_passes=False),
        scratch_types=[
            pltpu.VMEM((B_PER_W,), jnp.int32),
            pltpu.VMEM((PAIRS,), jnp.int32),
            pltpu.VMEM((E_PER_W,), jnp.int32),
            pltpu.VMEM((E_PER_W,), jnp.float32),
            pltpu.SemaphoreType.DMA,
        ],
    )
    def k(f_hbm, ids_hbm, gk_hbm, o_hbm, ids_v, gk_v, idx_v, val_v, sem):
        wid = lax.axis_index("s") * NC + lax.axis_index("c")
        base = wid * B_PER_W
        pltpu.sync_copy(ids_hbm.at[pl.ds(base, B_PER_W)], ids_v)
        pltpu.sync_copy(gk_hbm, gk_v)
        gk_row = gk_v[...]

        @pl.loop(0, B_PER_W)
        def _(j):
            vj = plsc.load_gather(
                ids_v, [jnp.full((PAIRS,), j, jnp.int32)])
            vterm = ((vj >> 7) << 10) + (vj & 127)
            idx_v[pl.ds(j * PAIRS, PAIRS)] = gk_row + vterm

        pltpu.async_copy(f_hbm.at[idx_v], val_v, sem).wait()
        pltpu.sync_copy(val_v, o_hbm.at[pl.ds(wid * E_PER_W, E_PER_W)])

    return k(flat, ids, gk)


def _unpack(p_u32):
    lo = lax.bitcast_convert_type(p_u32 << 16, jnp.float32)
    hi = lax.bitcast_convert_type(p_u32 & jnp.uint32(0xFFFF0000), jnp.float32)
    return lo.astype(jnp.bfloat16), hi.astype(jnp.bfloat16)


def _mlp_body(u_ref, r_ref, w1ul_ref, w1uh_ref, w1rl_ref, w1rh_ref, b1_ref,
              w2_ref, b2_ref, w3_ref, b3_ref, o_ref):
    u_lo, u_hi = _unpack(lax.bitcast_convert_type(u_ref[...], jnp.uint32))
    r_lo, r_hi = _unpack(lax.bitcast_convert_type(r_ref[...], jnp.uint32))
    f32 = jnp.float32
    h = jnp.dot(u_lo, w1ul_ref[...], preferred_element_type=f32)
    h += jnp.dot(u_hi, w1uh_ref[...], preferred_element_type=f32)
    h += jnp.dot(r_lo, w1rl_ref[...], preferred_element_type=f32)
    h += jnp.dot(r_hi, w1rh_ref[...], preferred_element_type=f32)
    h = jnp.maximum(h + b1_ref[...], 0.0)
    h = jnp.dot(h, w2_ref[...], preferred_element_type=f32)
    h = jnp.maximum(h + b2_ref[...], 0.0)
    o_ref[...] = (
        jnp.dot(h, w3_ref[...], preferred_element_type=f32) + b3_ref[...]
    )


def _tc_mlp(u, r, W1, b1, W2, b2, W3, b3):
    bf16 = jnp.bfloat16
    w1u = W1[:DIM]
    w1r = W1[DIM:]
    w1ul = w1u[jnp.array(PERM_LO)].astype(bf16)
    w1uh = w1u[jnp.array(PERM_HI)].astype(bf16)
    w1rl = w1r[jnp.array(PERM_LO)].astype(bf16)
    w1rh = w1r[jnp.array(PERM_HI)].astype(bf16)
    b1r = b1.reshape(1, -1)
    b2r = b2.reshape(1, -1)
    b3r = b3.reshape(1, -1)
    const = lambda shape: pl.BlockSpec(shape, lambda i: (0, 0))
    return pl.pallas_call(
        _mlp_body,
        grid=(B // MLP_BLOCK,),
        in_specs=[
            pl.BlockSpec((MLP_BLOCK, PAIRS), lambda i: (i, 0)),
            pl.BlockSpec((MLP_BLOCK, PAIRS), lambda i: (i, 0)),
            const(w1ul.shape),
            const(w1uh.shape),
            const(w1rl.shape),
            const(w1rh.shape),
            const(b1r.shape),
            const(W2.shape),
            const(b2r.shape),
            const(W3.shape),
            const(b3r.shape),
        ],
        out_specs=pl.BlockSpec((MLP_BLOCK, 1), lambda i: (i, 0)),
        out_shape=jax.ShapeDtypeStruct((B, 1), jnp.float32),
        compiler_params=pltpu.CompilerParams(
            dimension_semantics=("parallel",)),
    )(u, r, w1ul, w1uh, w1rl, w1rh, b1r, W2, b2r, W3, b3r)


def _pair_idx(ids, pt):
    i = jnp.arange(PAIRS, dtype=jnp.int32)[None, :]
    gk = (i // 8) * (pt * 1024) + (i % 8) * 128
    v = ids.astype(jnp.int32)[:, None]
    vterm = (v // 128) * 1024 + v % 128
    return (gk + vterm).reshape(-1)


def kernel(user_id, recipe_id, user_table, recipe_table, W1, b1, W2, b2, W3, b3):
    rflat = _flatten_pack(recipe_table.T, RPT, RK).reshape(-1)
    ro = _sc_gather(rflat, recipe_id.astype(jnp.int32), RPT)
    user_tableT, _ = lax.optimization_barrier((user_table.T, rflat))
    uflat = _flatten_pack(user_tableT, UPT, UK).reshape(-1)
    uo = _sc_gather(uflat, user_id.astype(jnp.int32), UPT)
    u = uo.reshape(B, PAIRS)
    r = ro.reshape(B, PAIRS)
    return _tc_mlp(u, r, W1, b1, W2, b2, W3, b3)


# speedup vs baseline: 17.7789x; 1.0383x over previous
import functools

import jax
import jax.numpy as jnp
from jax import lax
from jax.experimental import pallas as pl
from jax.experimental.pallas import tpu as pltpu
from jax.experimental.pallas import tpu_sc as plsc

B = 16384
DIM = 32
PAIRS = DIM // 2
NC, NS = 2, 16
NW = NC * NS
B_PER_W = B // NW
E_PER_W = B_PER_W * PAIRS
MLP_BLOCK = 4096

UV = 1000001
RV = 100001
UPT = 7813
RPT = 782
UK = 601
RK = 391

PERM_LO = [16 * (i // 8) + i % 8 for i in range(PAIRS)]
PERM_HI = [16 * (i // 8) + i % 8 + 8 for i in range(PAIRS)]


def _flatten_pack_body(in_ref, out_ref):
    lo = in_ref[:8, :].astype(jnp.bfloat16)
    hi = in_ref[8:, :].astype(jnp.bfloat16)
    lo_u = lax.bitcast_convert_type(lo, jnp.uint16).astype(jnp.uint32)
    hi_u = lax.bitcast_convert_type(hi, jnp.uint16).astype(jnp.uint32)
    packed = lax.bitcast_convert_type(lo_u | (hi_u << 16), jnp.float32)
    out_ref[...] = pltpu.einshape(
        "s(tc)->(ts)c", packed, c=128, assert_is_tile_preserving=True
    )


def _flatten_pack(tableT, pt, k):
    ngroups = DIM // 16
    out_rows = ngroups * 8 * pt
    return pl.pallas_call(
        _flatten_pack_body,
        grid=(ngroups, pt // k),
        in_specs=[pl.BlockSpec((16, k * 128), lambda g, j: (g, j))],
        out_specs=pl.BlockSpec((k * 8, 128), lambda g, j, _pt=pt, _k=k:
                               (g * (_pt // _k) + j, 0)),
        out_shape=jax.ShapeDtypeStruct((out_rows, 128), jnp.float32),
        compiler_params=pltpu.CompilerParams(
            dimension_semantics=("parallel", "arbitrary")),
    )(tableT)


def _sc_gather(flat, ids, pt):
    gk = jnp.array(
        [(i // 8) * (pt * 1024) + (i % 8) * 128 for i in range(PAIRS)],
        dtype=jnp.int32,
    )

    @functools.partial(
        pl.kernel,
        mesh=plsc.VectorSubcoreMesh(core_axis_name="c", subcore_axis_name="s"),
        out_type=jax.ShapeDtypeStruct((B * PAIRS,), jnp.float32),
        scratch_types=[
            pltpu.VMEM((B_PER_W,), jnp.int32),
            pltpu.VMEM((PAIRS,), jnp.int32),
            pltpu.VMEM((E_PER_W,), jnp.int32),
            pltpu.VMEM((E_PER_W,), jnp.float32),
            pltpu.SemaphoreType.DMA,
        ],
    )
    def k(f_hbm, ids_hbm, gk_hbm, o_hbm, ids_v, gk_v, idx_v, val_v, sem):
        wid = lax.axis_index("s") * NC + lax.axis_index("c")
        base = wid * B_PER_W
        pltpu.sync_copy(ids_hbm.at[pl.ds(base, B_PER_W)], ids_v)
        pltpu.sync_copy(gk_hbm, gk_v)
        gk_row = gk_v[...]

        @plsc.parallel_loop(0, B_PER_W, unroll=4)
        def _(j):
            vj = ids_v[pl.ds(j, 1)]
            vterm = ((vj >> 7) << 10) + (vj & 127)
            idx_v[pl.ds(j * PAIRS, PAIRS)] = gk_row + vterm

        pltpu.async_copy(f_hbm.at[idx_v], val_v, sem).wait()
        pltpu.sync_copy(val_v, o_hbm.at[pl.ds(wid * E_PER_W, E_PER_W)])

    return k(flat, ids, gk)


def _unpack(p_u32):
    lo = lax.bitcast_convert_type(p_u32 << 16, jnp.float32)
    hi = lax.bitcast_convert_type(p_u32 & jnp.uint32(0xFFFF0000), jnp.float32)
    return lo.astype(jnp.bfloat16), hi.astype(jnp.bfloat16)


def _mlp_body(u_ref, r_ref, w1ul_ref, w1uh_ref, w1rl_ref, w1rh_ref, b1_ref,
              w2_ref, b2_ref, w3_ref, b3_ref, o_ref):
    u_lo, u_hi = _unpack(lax.bitcast_convert_type(u_ref[...], jnp.uint32))
    r_lo, r_hi = _unpack(lax.bitcast_convert_type(r_ref[...], jnp.uint32))
    f32 = jnp.float32
    h = jnp.dot(u_lo, w1ul_ref[...], preferred_element_type=f32)
    h += jnp.dot(u_hi, w1uh_ref[...], preferred_element_type=f32)
    h += jnp.dot(r_lo, w1rl_ref[...], preferred_element_type=f32)
    h += jnp.dot(r_hi, w1rh_ref[...], preferred_element_type=f32)
    h = jnp.maximum(h + b1_ref[...], 0.0)
    h = jnp.dot(h, w2_ref[...], preferred_element_type=f32)
    h = jnp.maximum(h + b2_ref[...], 0.0)
    o_ref[...] = (
        jnp.dot(h, w3_ref[...], preferred_element_type=f32) + b3_ref[...]
    )


def _tc_mlp(u, r, W1, b1, W2, b2, W3, b3):
    bf16 = jnp.bfloat16
    w1u = W1[:DIM]
    w1r = W1[DIM:]
    w1ul = w1u[jnp.array(PERM_LO)].astype(bf16)
    w1uh = w1u[jnp.array(PERM_HI)].astype(bf16)
    w1rl = w1r[jnp.array(PERM_LO)].astype(bf16)
    w1rh = w1r[jnp.array(PERM_HI)].astype(bf16)
    b1r = b1.reshape(1, -1)
    b2r = b2.reshape(1, -1)
    b3r = b3.reshape(1, -1)
    const = lambda shape: pl.BlockSpec(shape, lambda i: (0, 0))
    return pl.pallas_call(
        _mlp_body,
        grid=(B // MLP_BLOCK,),
        in_specs=[
            pl.BlockSpec((MLP_BLOCK, PAIRS), lambda i: (i, 0)),
            pl.BlockSpec((MLP_BLOCK, PAIRS), lambda i: (i, 0)),
            const(w1ul.shape),
            const(w1uh.shape),
            const(w1rl.shape),
            const(w1rh.shape),
            const(b1r.shape),
            const(W2.shape),
            const(b2r.shape),
            const(W3.shape),
            const(b3r.shape),
        ],
        out_specs=pl.BlockSpec((MLP_BLOCK, 1), lambda i: (i, 0)),
        out_shape=jax.ShapeDtypeStruct((B, 1), jnp.float32),
        compiler_params=pltpu.CompilerParams(
            dimension_semantics=("parallel",)),
    )(u, r, w1ul, w1uh, w1rl, w1rh, b1r, W2, b2r, W3, b3r)


def _pair_idx(ids, pt):
    i = jnp.arange(PAIRS, dtype=jnp.int32)[None, :]
    gk = (i // 8) * (pt * 1024) + (i % 8) * 128
    v = ids.astype(jnp.int32)[:, None]
    vterm = (v // 128) * 1024 + v % 128
    return (gk + vterm).reshape(-1)


def kernel(user_id, recipe_id, user_table, recipe_table, W1, b1, W2, b2, W3, b3):
    rflat = _flatten_pack(recipe_table.T, RPT, RK).reshape(-1)
    ro = _sc_gather(rflat, recipe_id.astype(jnp.int32), RPT)
    user_tableT, _ = lax.optimization_barrier((user_table.T, rflat))
    uflat = _flatten_pack(user_tableT, UPT, UK).reshape(-1)
    uo = _sc_gather(uflat, user_id.astype(jnp.int32), UPT)
    u = uo.reshape(B, PAIRS)
    r = ro.reshape(B, PAIRS)
    return _tc_mlp(u, r, W1, b1, W2, b2, W3, b3)
